# Initial kernel scaffold; baseline (speedup 1.0000x reference)
import jax
import jax.numpy as jnp
from jax.experimental import pallas as pl


def kernel(idx, targets, table):
    raise NotImplementedError("write your pallas kernel here")


# speedup vs baseline: 1.6117x; 1.6117x over previous
import functools

import jax
import jax.numpy as jnp
from jax import lax
from jax.experimental import pallas as pl
from jax.experimental.pallas import tpu as pltpu
from jax.experimental.pallas import tpu_sc as plsc

VOCAB = 1000
N_ROWS = 1024 * 50
NC, NS, L = 2, 16, 16
NW = NC * NS
ROWS_PER_W = N_ROWS // NW
CHUNK = 32
NCHUNK = ROWS_PER_W // CHUNK


def _lse_body(table_ref, lse_ref):
    t = table_ref[...]
    m = jnp.max(t, axis=1)
    s = jnp.sum(jnp.exp(t - m[:, None]), axis=1)
    lse_ref[...] = m + jnp.log(s)


def _compute_lse(table):
    return pl.pallas_call(
        _lse_body,
        out_shape=jax.ShapeDtypeStruct((VOCAB,), jnp.float32),
    )(table)


def _sc_body(idx_hbm, tgt_hbm, table_hbm, lse_hbm,
             out_hbm, part_hbm,
             idx_v, tgt_v, lse_v, rows_v, acc_v, sem):
    wid = lax.axis_index("s") * NC + lax.axis_index("c")
    base = wid * ROWS_PER_W

    pltpu.sync_copy(idx_hbm.at[pl.ds(base, ROWS_PER_W)], idx_v)
    pltpu.sync_copy(tgt_hbm.at[pl.ds(base, ROWS_PER_W)], tgt_v)
    pltpu.sync_copy(lse_hbm, lse_v.at[pl.ds(0, VOCAB)])

    lanes = lax.iota(jnp.int32, L)

    def chunk_body(c, acc):
        off = c * CHUNK
        pltpu.async_copy(table_hbm.at[idx_v.at[pl.ds(off, CHUNK)]],
                         rows_v.at[pl.ds(0, CHUNK)], sem).wait()
        pltpu.sync_copy(rows_v.at[pl.ds(0, CHUNK)],
                        out_hbm.at[pl.ds(base + off, CHUNK)])

        for g in range(CHUNK // L):
            i16 = idx_v[pl.ds(off + g * L, L)]
            t16 = tgt_v[pl.ds(off + g * L, L)]
            for j in range(L):
                i_s = i16[j]
                t_s = t16[j]
                i0 = jnp.minimum(i_s, VOCAB - L)
                t0 = jnp.minimum(t_s, VOCAB - L)
                lv = lse_v[pl.ds(i0, L)]
                rv = rows_v[g * L + j, pl.ds(t0, L)]
                acc = acc + jnp.where(lanes == i_s - i0, lv, 0.0)
                acc = acc - jnp.where(lanes == t_s - t0, rv, 0.0)
        return acc

    acc = lax.fori_loop(0, NCHUNK, chunk_body, jnp.zeros((L,), jnp.float32))
    acc_v[...] = acc
    pltpu.sync_copy(acc_v, part_hbm.at[wid])


_sc_gather = functools.partial(
    pl.kernel,
    out_type=[
        jax.ShapeDtypeStruct((N_ROWS, VOCAB), jnp.float32),
        jax.ShapeDtypeStruct((NW, L), jnp.float32),
    ],
    mesh=plsc.VectorSubcoreMesh(core_axis_name="c", subcore_axis_name="s"),
    compiler_params=pltpu.CompilerParams(use_tc_tiling_on_sc=False),
    scratch_types=[
        pltpu.VMEM((ROWS_PER_W,), jnp.int32),
        pltpu.VMEM((ROWS_PER_W,), jnp.int32),
        pltpu.VMEM((VOCAB + L,), jnp.float32),
        pltpu.VMEM((CHUNK + 1, VOCAB), jnp.float32),
        pltpu.VMEM((L,), jnp.float32),
        pltpu.SemaphoreType.DMA,
    ],
)(_sc_body)


@jax.jit
def kernel(idx, targets, table):
    lse = _compute_lse(table)
    idx_flat = idx.reshape(N_ROWS)
    tgt_flat = targets.reshape(N_ROWS)
    logits2d, partials = _sc_gather(idx_flat, tgt_flat, table, lse)
    loss = jnp.sum(partials) / N_ROWS
    return (logits2d, loss)


# speedup vs baseline: 1.6902x; 1.0487x over previous
import functools

import jax
import jax.numpy as jnp
from jax import lax
from jax.experimental import pallas as pl
from jax.experimental.pallas import tpu as pltpu
from jax.experimental.pallas import tpu_sc as plsc

VOCAB = 1000
N_ROWS = 1024 * 50
NC, NS, L = 2, 16, 16
NW = NC * NS
ROWS_PER_W = N_ROWS // NW
CHUNK = 32
NCHUNK = ROWS_PER_W // CHUNK


def _lse_body(table_ref, lse_ref):
    t = table_ref[...]
    m = jnp.max(t, axis=1)
    s = jnp.sum(jnp.exp(t - m[:, None]), axis=1)
    lse_ref[...] = m + jnp.log(s)


def _compute_lse(table):
    return pl.pallas_call(
        _lse_body,
        out_shape=jax.ShapeDtypeStruct((VOCAB,), jnp.float32),
    )(table)


def _sc_body(idx_hbm, tgt_hbm, table_hbm, lse_hbm,
             out_hbm, part_hbm,
             idx_v, tgt_v, lse_v, rows_a, rows_b, acc_v,
             gsem_a, gsem_b, ssem_a, ssem_b):
    wid = lax.axis_index("s") * NC + lax.axis_index("c")
    base = wid * ROWS_PER_W

    pltpu.sync_copy(idx_hbm.at[pl.ds(base, ROWS_PER_W)], idx_v)
    pltpu.sync_copy(tgt_hbm.at[pl.ds(base, ROWS_PER_W)], tgt_v)
    pltpu.sync_copy(lse_hbm, lse_v.at[pl.ds(0, VOCAB)])

    lanes = lax.iota(jnp.int32, L)

    def start_gather(off, buf, sem):
        pltpu.async_copy(table_hbm.at[idx_v.at[pl.ds(off, CHUNK)]],
                         buf.at[pl.ds(0, CHUNK)], sem)

    def wait_gather(buf, sem):
        pltpu.make_async_copy(table_hbm.at[pl.ds(0, CHUNK)],
                              buf.at[pl.ds(0, CHUNK)], sem).wait()

    def start_scatter(off, buf, sem):
        pltpu.async_copy(buf.at[pl.ds(0, CHUNK)],
                         out_hbm.at[pl.ds(base + off, CHUNK)], sem)

    def wait_scatter(buf, sem):
        pltpu.make_async_copy(buf.at[pl.ds(0, CHUNK)],
                              out_hbm.at[pl.ds(base, CHUNK)], sem).wait()

    def loss_chunk(buf, off, acc):
        for g in range(CHUNK // L):
            i16 = idx_v[pl.ds(off + g * L, L)]
            t16 = tgt_v[pl.ds(off + g * L, L)]
            for j in range(L):
                i_s = i16[j]
                t_s = t16[j]
                i0 = jnp.minimum(i_s, VOCAB - L)
                t0 = jnp.minimum(t_s, VOCAB - L)
                lv = lse_v[pl.ds(i0, L)]
                rv = buf[g * L + j, pl.ds(t0, L)]
                acc = acc + jnp.where(lanes == i_s - i0, lv, 0.0)
                acc = acc - jnp.where(lanes == t_s - t0, rv, 0.0)
        return acc

    start_gather(0, rows_a, gsem_a)

    def super_body(s, acc):
        off_a = (2 * s) * CHUNK
        off_b = off_a + CHUNK
        start_gather(off_b, rows_b, gsem_b)
        wait_gather(rows_a, gsem_a)
        acc = loss_chunk(rows_a, off_a, acc)
        start_scatter(off_a, rows_a, ssem_a)
        wait_gather(rows_b, gsem_b)
        acc = loss_chunk(rows_b, off_b, acc)
        start_scatter(off_b, rows_b, ssem_b)
        wait_scatter(rows_a, ssem_a)

        @pl.when(s < NCHUNK // 2 - 1)
        def _():
            start_gather(off_a + 2 * CHUNK, rows_a, gsem_a)

        wait_scatter(rows_b, ssem_b)
        return acc

    acc = lax.fori_loop(0, NCHUNK // 2, super_body,
                        jnp.zeros((L,), jnp.float32))
    acc_v[...] = acc
    pltpu.sync_copy(acc_v, part_hbm.at[wid])


_sc_gather = functools.partial(
    pl.kernel,
    out_type=[
        jax.ShapeDtypeStruct((N_ROWS, VOCAB), jnp.float32),
        jax.ShapeDtypeStruct((NW, L), jnp.float32),
    ],
    mesh=plsc.VectorSubcoreMesh(core_axis_name="c", subcore_axis_name="s"),
    compiler_params=pltpu.CompilerParams(use_tc_tiling_on_sc=False),
    scratch_types=[
        pltpu.VMEM((ROWS_PER_W,), jnp.int32),
        pltpu.VMEM((ROWS_PER_W,), jnp.int32),
        pltpu.VMEM((VOCAB + L,), jnp.float32),
        pltpu.VMEM((CHUNK + 1, VOCAB), jnp.float32),
        pltpu.VMEM((CHUNK + 1, VOCAB), jnp.float32),
        pltpu.VMEM((L,), jnp.float32),
        pltpu.SemaphoreType.DMA,
        pltpu.SemaphoreType.DMA,
        pltpu.SemaphoreType.DMA,
        pltpu.SemaphoreType.DMA,
    ],
)(_sc_body)


@jax.jit
def kernel(idx, targets, table):
    lse = _compute_lse(table)
    idx_flat = idx.reshape(N_ROWS)
    tgt_flat = targets.reshape(N_ROWS)
    logits2d, partials = _sc_gather(idx_flat, tgt_flat, table, lse)
    loss = jnp.sum(partials) / N_ROWS
    return (logits2d, loss)


# speedup vs baseline: 2.7808x; 1.6453x over previous
import functools

import jax
import jax.numpy as jnp
from jax import lax
from jax.experimental import pallas as pl
from jax.experimental.pallas import tpu as pltpu
from jax.experimental.pallas import tpu_sc as plsc

VOCAB = 1000
VOCAB_P = 1024
N_ROWS = 1024 * 50
NC, NS, L = 2, 16, 16
NW = NC * NS
ROWS_PER_W = N_ROWS // NW
CHUNK = 32
NCHUNK = ROWS_PER_W // CHUNK


def _lse_body(table_ref, lse_ref):
    t = table_ref[...]
    m = jnp.max(t, axis=1)
    s = jnp.sum(jnp.exp(t - m[:, None]), axis=1)
    lse_ref[...] = m + jnp.log(s)


def _compute_lse(table):
    return pl.pallas_call(
        _lse_body,
        out_shape=jax.ShapeDtypeStruct((VOCAB,), jnp.float32),
    )(table)


def _sc_body(idx_hbm, tgt_hbm, table_hbm, lse_hbm,
             out_hbm, part_hbm,
             idx_v, tgt_v, lse_v, rows_a, rows_b, acc_v,
             gsem_a, gsem_b, ssem_a, ssem_b):
    wid = lax.axis_index("s") * NC + lax.axis_index("c")
    base = wid * ROWS_PER_W

    pltpu.sync_copy(idx_hbm.at[pl.ds(base, ROWS_PER_W)], idx_v)
    pltpu.sync_copy(tgt_hbm.at[pl.ds(base, ROWS_PER_W)], tgt_v)
    pltpu.sync_copy(lse_hbm, lse_v.at[pl.ds(0, VOCAB)])

    lanes = lax.iota(jnp.int32, L)

    def start_gather(off, buf, sem):
        pltpu.async_copy(table_hbm.at[idx_v.at[pl.ds(off, CHUNK)]],
                         buf.at[pl.ds(0, CHUNK)], sem)

    def wait_gather(buf, sem):
        pltpu.make_async_copy(table_hbm.at[pl.ds(0, CHUNK)],
                              buf.at[pl.ds(0, CHUNK)], sem).wait()

    def start_scatter(off, buf, sem):
        pltpu.async_copy(buf.at[pl.ds(0, CHUNK)],
                         out_hbm.at[pl.ds(base + off, CHUNK)], sem)

    def wait_scatter(buf, sem):
        pltpu.make_async_copy(buf.at[pl.ds(0, CHUNK)],
                              out_hbm.at[pl.ds(base, CHUNK)], sem).wait()

    def loss_chunk(buf, off, acc):
        for g in range(CHUNK // L):
            i16 = idx_v[pl.ds(off + g * L, L)]
            t16 = tgt_v[pl.ds(off + g * L, L)]
            for j in range(L):
                i_s = i16[j]
                t_s = t16[j]
                i0 = (i_s // L) * L
                t0 = (t_s // L) * L
                lv = lse_v[pl.ds(i0, L)]
                rv = buf[g * L + j, pl.ds(t0, L)]
                acc = acc + jnp.where(lanes == i_s - i0, lv, 0.0)
                acc = acc - jnp.where(lanes == t_s - t0, rv, 0.0)
        return acc

    start_gather(0, rows_a, gsem_a)

    def super_body(s, acc):
        off_a = (2 * s) * CHUNK
        off_b = off_a + CHUNK
        start_gather(off_b, rows_b, gsem_b)
        wait_gather(rows_a, gsem_a)
        acc = loss_chunk(rows_a, off_a, acc)
        start_scatter(off_a, rows_a, ssem_a)
        wait_gather(rows_b, gsem_b)
        acc = loss_chunk(rows_b, off_b, acc)
        start_scatter(off_b, rows_b, ssem_b)
        wait_scatter(rows_a, ssem_a)

        @pl.when(s < NCHUNK // 2 - 1)
        def _():
            start_gather(off_a + 2 * CHUNK, rows_a, gsem_a)

        wait_scatter(rows_b, ssem_b)
        return acc

    acc = lax.fori_loop(0, NCHUNK // 2, super_body,
                        jnp.zeros((L,), jnp.float32))
    acc_v[...] = acc
    pltpu.sync_copy(acc_v, part_hbm.at[wid])


_sc_gather = functools.partial(
    pl.kernel,
    out_type=[
        jax.ShapeDtypeStruct((N_ROWS, VOCAB_P), jnp.float32),
        jax.ShapeDtypeStruct((NW, L), jnp.float32),
    ],
    mesh=plsc.VectorSubcoreMesh(core_axis_name="c", subcore_axis_name="s"),
    scratch_types=[
        pltpu.VMEM((ROWS_PER_W,), jnp.int32),
        pltpu.VMEM((ROWS_PER_W,), jnp.int32),
        pltpu.VMEM((VOCAB + L,), jnp.float32),
        pltpu.VMEM((CHUNK, VOCAB_P), jnp.float32),
        pltpu.VMEM((CHUNK, VOCAB_P), jnp.float32),
        pltpu.VMEM((L,), jnp.float32),
        pltpu.SemaphoreType.DMA,
        pltpu.SemaphoreType.DMA,
        pltpu.SemaphoreType.DMA,
        pltpu.SemaphoreType.DMA,
    ],
)(_sc_body)


@jax.jit
def kernel(idx, targets, table):
    lse = _compute_lse(table)
    idx_flat = idx.reshape(N_ROWS)
    tgt_flat = targets.reshape(N_ROWS)
    table_p = jnp.pad(table, ((0, 0), (0, VOCAB_P - VOCAB)))
    padded, partials = _sc_gather(idx_flat, tgt_flat, table_p, lse)
    logits2d = lax.slice(padded, (0, 0), (N_ROWS, VOCAB))
    loss = jnp.sum(partials) / N_ROWS
    return (logits2d, loss)
